# baseline (speedup 1.0000x reference)
import jax
import jax.numpy as jnp
from jax import lax
from jax.experimental import pallas as pl
from jax.experimental.pallas import tpu as pltpu
from jax.experimental.pallas import tpu_sc as plsc

N_NODES = 1000000
D = 128
B = 16384
L = 16
NC = 2
NS = 16
NW = NC * NS
ROWS_PER_W = B // NW
CHUNK_SIZES = (192, 160, 128, 32)
CHUNK_OFFS = (0, 192, 352, 480)
CHUNKS = len(CHUNK_SIZES)
UNROLL = 4

_GATHER_DNUMS = lax.GatherDimensionNumbers(
    offset_dims=(), collapsed_slice_dims=(0,), start_index_map=(0,))


def _sc_body(mem_hbm, idx_hbm, td_hbm, w_hbm, b_hbm, out_hbm,
             idx_v, td_v, w_v, b_v, rows_v, gsems, wsems):
    wid = lax.axis_index("s") * NC + lax.axis_index("c")
    base = wid * ROWS_PER_W

    pltpu.sync_copy(idx_hbm.at[pl.ds(base, ROWS_PER_W)], idx_v)
    pltpu.sync_copy(td_hbm.at[pl.ds(base, ROWS_PER_W)], td_v)
    pltpu.sync_copy(w_hbm, w_v)
    pltpu.sync_copy(b_hbm, b_v)
    gathers = [
        pltpu.async_copy(
            mem_hbm.at[idx_v.at[pl.ds(CHUNK_OFFS[k], CHUNK_SIZES[k])]],
            rows_v.at[pl.ds(CHUNK_OFFS[k], CHUNK_SIZES[k])], gsems[k])
        for k in range(CHUNKS)
    ]

    w_c = [w_v[pl.ds(c * L, L)] for c in range(D // L)]
    b1_c = [b_v[pl.ds(c * L, L)] + 1.0 for c in range(D // L)]

    def row_body(i, carry):
        blk = i * UNROLL // L
        tdv = td_v[pl.ds(blk * L, L)]
        for u in range(UNROLL):
            r = i * UNROLL + u
            lane = r - blk * L
            lidx = jnp.broadcast_to(
                jnp.reshape(lane.astype(jnp.int32), (1, 1)), (L, 1))
            tds = lax.gather(
                tdv, lidx, _GATHER_DNUMS, (1,),
                mode=lax.GatherScatterMode.PROMISE_IN_BOUNDS)
            for c in range(D // L):
                sl = pl.ds(c * L, L)
                rows_v[r, sl] = rows_v[r, sl] * (tds * w_c[c] + b1_c[c])
        return carry

    writes = []
    for k in range(CHUNKS):
        gathers[k].wait()
        lax.fori_loop(CHUNK_OFFS[k] // UNROLL,
                      (CHUNK_OFFS[k] + CHUNK_SIZES[k]) // UNROLL,
                      row_body, 0)
        writes.append(
            pltpu.async_copy(
                rows_v.at[pl.ds(CHUNK_OFFS[k], CHUNK_SIZES[k])],
                out_hbm.at[pl.ds(base + CHUNK_OFFS[k], CHUNK_SIZES[k])],
                wsems[k]))

    for wr in writes:
        wr.wait()


@jax.jit
def _time_embedding_sc(memory, source_nodes, time_diffs, W, b):
    mesh = plsc.VectorSubcoreMesh(
        core_axis_name="c", subcore_axis_name="s",
        num_cores=NC, num_subcores=NS)
    return pl.kernel(
        _sc_body,
        out_type=jax.ShapeDtypeStruct((B, D), jnp.float32),
        mesh=mesh,
        scratch_types=[
            pltpu.VMEM((ROWS_PER_W,), jnp.int32),
            pltpu.VMEM((ROWS_PER_W,), jnp.float32),
            pltpu.VMEM((D,), jnp.float32),
            pltpu.VMEM((D,), jnp.float32),
            pltpu.VMEM((ROWS_PER_W, D), jnp.float32),
            [pltpu.SemaphoreType.DMA] * CHUNKS,
            [pltpu.SemaphoreType.DMA] * CHUNKS,
        ],
    )(memory, source_nodes, time_diffs, W, b)


def kernel(memory, source_nodes, timestamps, n_layers, n_neighbors,
           time_diffs, W, b):
    return _time_embedding_sc(memory, source_nodes, time_diffs, W[:, 0], b)


# speedup vs baseline: 1.0756x; 1.0756x over previous
import jax
import jax.numpy as jnp
from jax import lax
from jax.experimental import pallas as pl
from jax.experimental.pallas import tpu as pltpu
from jax.experimental.pallas import tpu_sc as plsc

N_NODES = 1000000
D = 128
B = 16384
L = 16
NC = 2
NS = 16
NW = NC * NS
ROWS_PER_W = B // NW
CHUNK_SIZES = (192, 160, 128, 32)
CHUNK_OFFS = (0, 192, 352, 480)
CHUNKS = len(CHUNK_SIZES)
UNROLL = 4

_GATHER_DNUMS = lax.GatherDimensionNumbers(
    offset_dims=(), collapsed_slice_dims=(0,), start_index_map=(0,))


def _sc_body(mem_hbm, idx_hbm, td_hbm, w_hbm, b_hbm, out_hbm,
             idx_v, td_v, w_v, b_v, rows_v, gsems, wsems):
    wid = lax.axis_index("s") * NC + lax.axis_index("c")
    base = wid * ROWS_PER_W

    pltpu.sync_copy(idx_hbm.at[pl.ds(base, ROWS_PER_W)], idx_v)
    gathers = [
        pltpu.async_copy(
            mem_hbm.at[idx_v.at[pl.ds(CHUNK_OFFS[k], CHUNK_SIZES[k])]],
            rows_v.at[pl.ds(CHUNK_OFFS[k], CHUNK_SIZES[k])], gsems[k])
        for k in range(CHUNKS)
    ]

    pltpu.sync_copy(td_hbm.at[pl.ds(base, ROWS_PER_W)], td_v)
    pltpu.sync_copy(w_hbm, w_v)
    pltpu.sync_copy(b_hbm, b_v)

    w_c = [w_v[pl.ds(c * L, L)] for c in range(D // L)]
    b1_c = [b_v[pl.ds(c * L, L)] + 1.0 for c in range(D // L)]

    def row_body(i, carry):
        blk = i * UNROLL // L
        tdv = td_v[pl.ds(blk * L, L)]
        for u in range(UNROLL):
            r = i * UNROLL + u
            lane = r - blk * L
            lidx = jnp.broadcast_to(
                jnp.reshape(lane.astype(jnp.int32), (1, 1)), (L, 1))
            tds = lax.gather(
                tdv, lidx, _GATHER_DNUMS, (1,),
                mode=lax.GatherScatterMode.PROMISE_IN_BOUNDS)
            for c in range(D // L):
                sl = pl.ds(c * L, L)
                rows_v[r, sl] = rows_v[r, sl] * (tds * w_c[c] + b1_c[c])
        return carry

    writes = []
    for k in range(CHUNKS):
        gathers[k].wait()
        lax.fori_loop(CHUNK_OFFS[k] // UNROLL,
                      (CHUNK_OFFS[k] + CHUNK_SIZES[k]) // UNROLL,
                      row_body, 0)
        writes.append(
            pltpu.async_copy(
                rows_v.at[pl.ds(CHUNK_OFFS[k], CHUNK_SIZES[k])],
                out_hbm.at[pl.ds(base + CHUNK_OFFS[k], CHUNK_SIZES[k])],
                wsems[k]))

    for wr in writes:
        wr.wait()


@jax.jit
def _time_embedding_sc(memory, source_nodes, time_diffs, W, b):
    mesh = plsc.VectorSubcoreMesh(
        core_axis_name="c", subcore_axis_name="s",
        num_cores=NC, num_subcores=NS)
    return pl.kernel(
        _sc_body,
        out_type=jax.ShapeDtypeStruct((B, D), jnp.float32),
        mesh=mesh,
        scratch_types=[
            pltpu.VMEM((ROWS_PER_W,), jnp.int32),
            pltpu.VMEM((ROWS_PER_W,), jnp.float32),
            pltpu.VMEM((D,), jnp.float32),
            pltpu.VMEM((D,), jnp.float32),
            pltpu.VMEM((ROWS_PER_W, D), jnp.float32),
            [pltpu.SemaphoreType.DMA] * CHUNKS,
            [pltpu.SemaphoreType.DMA] * CHUNKS,
        ],
    )(memory, source_nodes, time_diffs, W, b)


def kernel(memory, source_nodes, timestamps, n_layers, n_neighbors,
           time_diffs, W, b):
    return _time_embedding_sc(memory, source_nodes, time_diffs, W[:, 0], b)


# speedup vs baseline: 1.0829x; 1.0068x over previous
import jax
import jax.numpy as jnp
from jax import lax
from jax.experimental import pallas as pl
from jax.experimental.pallas import tpu as pltpu
from jax.experimental.pallas import tpu_sc as plsc

N_NODES = 1000000
D = 128
B = 16384
L = 16
NC = 2
NS = 16
NW = NC * NS
ROWS_PER_W = B // NW
CHUNK_SIZES = (192, 160, 128, 32)
CHUNK_OFFS = (0, 192, 352, 480)
CHUNKS = len(CHUNK_SIZES)
UNROLL = 4

_GATHER_DNUMS = lax.GatherDimensionNumbers(
    offset_dims=(), collapsed_slice_dims=(0,), start_index_map=(0,))


def _sc_body(mem_hbm, idx_hbm, td_hbm, w_hbm, b_hbm, out_hbm,
             idx_v, td_v, w_v, b_v, rows_v, gsems, wsems):
    wid = lax.axis_index("s") * NC + lax.axis_index("c")
    base = wid * ROWS_PER_W

    pltpu.sync_copy(idx_hbm.at[pl.ds(base, ROWS_PER_W)], idx_v)
    gathers = [
        pltpu.async_copy(
            mem_hbm.at[idx_v.at[pl.ds(CHUNK_OFFS[k], CHUNK_SIZES[k])]],
            rows_v.at[pl.ds(CHUNK_OFFS[k], CHUNK_SIZES[k])], gsems[k],
            priority=1)
        for k in range(CHUNKS)
    ]

    pltpu.sync_copy(td_hbm.at[pl.ds(base, ROWS_PER_W)], td_v)
    pltpu.sync_copy(w_hbm, w_v)
    pltpu.sync_copy(b_hbm, b_v)

    w_c = [w_v[pl.ds(c * L, L)] for c in range(D // L)]
    b1_c = [b_v[pl.ds(c * L, L)] + 1.0 for c in range(D // L)]

    def row_body(i, carry):
        blk = i * UNROLL // L
        tdv = td_v[pl.ds(blk * L, L)]
        for u in range(UNROLL):
            r = i * UNROLL + u
            lane = r - blk * L
            lidx = jnp.broadcast_to(
                jnp.reshape(lane.astype(jnp.int32), (1, 1)), (L, 1))
            tds = lax.gather(
                tdv, lidx, _GATHER_DNUMS, (1,),
                mode=lax.GatherScatterMode.PROMISE_IN_BOUNDS)
            for c in range(D // L):
                sl = pl.ds(c * L, L)
                rows_v[r, sl] = rows_v[r, sl] * (tds * w_c[c] + b1_c[c])
        return carry

    writes = []
    for k in range(CHUNKS):
        gathers[k].wait()
        lax.fori_loop(CHUNK_OFFS[k] // UNROLL,
                      (CHUNK_OFFS[k] + CHUNK_SIZES[k]) // UNROLL,
                      row_body, 0)
        writes.append(
            pltpu.async_copy(
                rows_v.at[pl.ds(CHUNK_OFFS[k], CHUNK_SIZES[k])],
                out_hbm.at[pl.ds(base + CHUNK_OFFS[k], CHUNK_SIZES[k])],
                wsems[k]))

    for wr in writes:
        wr.wait()


@jax.jit
def _time_embedding_sc(memory, source_nodes, time_diffs, W, b):
    mesh = plsc.VectorSubcoreMesh(
        core_axis_name="c", subcore_axis_name="s",
        num_cores=NC, num_subcores=NS)
    return pl.kernel(
        _sc_body,
        out_type=jax.ShapeDtypeStruct((B, D), jnp.float32),
        mesh=mesh,
        scratch_types=[
            pltpu.VMEM((ROWS_PER_W,), jnp.int32),
            pltpu.VMEM((ROWS_PER_W,), jnp.float32),
            pltpu.VMEM((D,), jnp.float32),
            pltpu.VMEM((D,), jnp.float32),
            pltpu.VMEM((ROWS_PER_W, D), jnp.float32),
            [pltpu.SemaphoreType.DMA] * CHUNKS,
            [pltpu.SemaphoreType.DMA] * CHUNKS,
        ],
    )(memory, source_nodes, time_diffs, W, b)


def kernel(memory, source_nodes, timestamps, n_layers, n_neighbors,
           time_diffs, W, b):
    return _time_embedding_sc(memory, source_nodes, time_diffs, W[:, 0], b)
